# baseline (speedup 1.0000x reference)
import functools

import jax
import jax.numpy as jnp
from jax import lax
from jax.experimental import pallas as pl
from jax.experimental.pallas import tpu as pltpu
from jax.experimental.pallas import tpu_sc as plsc

B = 16384
L = 200
D = 64
NC = 2
NS = 16
NW = NC * NS
RPW = B // NW
CB = 8
NCHUNK = RPW // CB
CB_L = CB * L
NG = D // 32
UNROLL = 2
HI_MASK = -65536


def _body(x_hbm, table_hbm, out_hbm, idx_v, rows_v, out_v, sem0, sem1):
    wid = lax.axis_index("s") * NC + lax.axis_index("c")
    row_base = wid * RPW
    sems = (sem0, sem1)

    def fire(slot, c):
        r0 = row_base + c * CB
        pltpu.sync_copy(x_hbm.at[pl.ds(r0, CB), :], idx_v.at[slot])
        for b in range(CB):
            pltpu.async_copy(
                table_hbm.at[idx_v.at[slot].at[b]],
                rows_v.at[slot].at[pl.ds(b * L, L)],
                sems[slot],
            )

    def drain(slot):
        for b in range(CB):
            pltpu.make_async_copy(
                table_hbm.at[idx_v.at[slot].at[b]],
                rows_v.at[slot].at[pl.ds(b * L, L)],
                sems[slot],
            ).wait()

    def reduce_store(slot, c):
        r0 = row_base + c * CB
        rows = rows_v.at[slot]
        for b in range(CB):
            def t_body(t, accs):
                base = b * L + UNROLL * t
                for u in range(UNROLL):
                    new = []
                    for g in range(NG):
                        ev, od = plsc.unpack(
                            rows[base + u, pl.ds(g * 32, 32)],
                            format=plsc.PackFormat.INTERLEAVED,
                        )
                        new.append(accs[2 * g] + ev)
                        new.append(accs[2 * g + 1] + od)
                    accs = tuple(new)
                return accs
            accs = lax.fori_loop(
                0, L // UNROLL, t_body,
                tuple(jnp.zeros((16,), jnp.float32) for _ in range(2 * NG)),
            )
            lane = lax.iota(jnp.int32, 16)
            brow = jnp.full((16,), b, jnp.int32)
            for g in range(NG):
                cols = lane * 2 + (32 * g)
                plsc.store_scatter(
                    out_v, [brow, cols], accs[2 * g] * jnp.float32(1.0 / L)
                )
                plsc.store_scatter(
                    out_v, [brow, cols + 1],
                    accs[2 * g + 1] * jnp.float32(1.0 / L),
                )
        pltpu.sync_copy(out_v, out_hbm.at[pl.ds(r0, CB), :])

    fire(0, 0)

    def pair_body(k, carry):
        c0 = 2 * k
        fire(1, c0 + 1)
        drain(0)
        reduce_store(0, c0)

        @pl.when(c0 + 2 < NCHUNK)
        def _():
            fire(0, c0 + 2)

        drain(1)
        reduce_store(1, c0 + 1)
        return carry

    lax.fori_loop(0, NCHUNK // 2, pair_body, 0)


@functools.partial(
    pl.kernel,
    mesh=plsc.VectorSubcoreMesh(core_axis_name="c", subcore_axis_name="s"),
    out_type=jax.ShapeDtypeStruct((B, D), jnp.float32),
    scratch_types=[
        pltpu.VMEM((2, CB, L), jnp.int32),
        pltpu.VMEM((2, CB_L, D), jnp.bfloat16),
        pltpu.VMEM((CB, D), jnp.float32),
        pltpu.SemaphoreType.DMA,
        pltpu.SemaphoreType.DMA,
    ],
    compiler_params=pltpu.CompilerParams(
        use_tc_tiling_on_sc=False, needs_layout_passes=False
    ),
)
def _pooled_lookup(x_hbm, table_hbm, out_hbm, idx_v, rows_v, out_v, sem0, sem1):
    _body(x_hbm, table_hbm, out_hbm, idx_v, rows_v, out_v, sem0, sem1)


@jax.jit
def kernel(x, table):
    return _pooled_lookup(x, table.astype(jnp.bfloat16))


# speedup vs baseline: 1.0348x; 1.0348x over previous
import functools

import jax
import jax.numpy as jnp
from jax import lax
from jax.experimental import pallas as pl
from jax.experimental.pallas import tpu as pltpu
from jax.experimental.pallas import tpu_sc as plsc

B = 16384
L = 200
D = 64
NC = 2
NS = 16
NW = NC * NS
RPW = B // NW
CB = 8
NCHUNK = RPW // CB
CB_L = CB * L
NG = D // 32
UNROLL = 4
HI_MASK = -65536


def _body(x_hbm, table_hbm, out_hbm, idx_v, rows_v, out_v, sem0, sem1):
    wid = lax.axis_index("s") * NC + lax.axis_index("c")
    row_base = wid * RPW
    sems = (sem0, sem1)

    def fire(slot, c):
        r0 = row_base + c * CB
        pltpu.sync_copy(x_hbm.at[pl.ds(r0, CB), :], idx_v.at[slot])
        for b in range(CB):
            pltpu.async_copy(
                table_hbm.at[idx_v.at[slot].at[b]],
                rows_v.at[slot].at[pl.ds(b * L, L)],
                sems[slot],
            )

    def drain(slot):
        for b in range(CB):
            pltpu.make_async_copy(
                table_hbm.at[idx_v.at[slot].at[b]],
                rows_v.at[slot].at[pl.ds(b * L, L)],
                sems[slot],
            ).wait()

    def reduce_store(slot, c):
        r0 = row_base + c * CB
        rows = rows_v.at[slot]
        for b in range(CB):
            def t_body(t, accs):
                base = b * L + UNROLL * t
                for u in range(UNROLL):
                    new = []
                    for g in range(NG):
                        ev, od = plsc.unpack(
                            rows[base + u, pl.ds(g * 32, 32)],
                            format=plsc.PackFormat.INTERLEAVED,
                        )
                        new.append(accs[2 * g] + ev)
                        new.append(accs[2 * g + 1] + od)
                    accs = tuple(new)
                return accs
            accs = lax.fori_loop(
                0, L // UNROLL, t_body,
                tuple(jnp.zeros((16,), jnp.float32) for _ in range(2 * NG)),
            )
            lane = lax.iota(jnp.int32, 16)
            brow = jnp.full((16,), b, jnp.int32)
            for g in range(NG):
                cols = lane * 2 + (32 * g)
                plsc.store_scatter(
                    out_v, [brow, cols], accs[2 * g] * jnp.float32(1.0 / L)
                )
                plsc.store_scatter(
                    out_v, [brow, cols + 1],
                    accs[2 * g + 1] * jnp.float32(1.0 / L),
                )
        pltpu.sync_copy(out_v, out_hbm.at[pl.ds(r0, CB), :])

    fire(0, 0)

    def pair_body(k, carry):
        c0 = 2 * k
        fire(1, c0 + 1)
        drain(0)
        reduce_store(0, c0)

        @pl.when(c0 + 2 < NCHUNK)
        def _():
            fire(0, c0 + 2)

        drain(1)
        reduce_store(1, c0 + 1)
        return carry

    lax.fori_loop(0, NCHUNK // 2, pair_body, 0)


@functools.partial(
    pl.kernel,
    mesh=plsc.VectorSubcoreMesh(core_axis_name="c", subcore_axis_name="s"),
    out_type=jax.ShapeDtypeStruct((B, D), jnp.float32),
    scratch_types=[
        pltpu.VMEM((2, CB, L), jnp.int32),
        pltpu.VMEM((2, CB_L, D), jnp.bfloat16),
        pltpu.VMEM((CB, D), jnp.float32),
        pltpu.SemaphoreType.DMA,
        pltpu.SemaphoreType.DMA,
    ],
    compiler_params=pltpu.CompilerParams(
        use_tc_tiling_on_sc=False, needs_layout_passes=False
    ),
)
def _pooled_lookup(x_hbm, table_hbm, out_hbm, idx_v, rows_v, out_v, sem0, sem1):
    _body(x_hbm, table_hbm, out_hbm, idx_v, rows_v, out_v, sem0, sem1)


@jax.jit
def kernel(x, table):
    return _pooled_lookup(x, table.astype(jnp.bfloat16))


# speedup vs baseline: 1.0379x; 1.0030x over previous
import functools

import jax
import jax.numpy as jnp
from jax import lax
from jax.experimental import pallas as pl
from jax.experimental.pallas import tpu as pltpu
from jax.experimental.pallas import tpu_sc as plsc

B = 16384
L = 200
D = 64
NC = 2
NS = 16
NW = NC * NS
RPW = B // NW
CB = 8
NCHUNK = RPW // CB
CB_L = CB * L
NG = D // 32
UNROLL = 8
HI_MASK = -65536


def _body(x_hbm, table_hbm, out_hbm, idx_v, rows_v, out_v, sem0, sem1):
    wid = lax.axis_index("s") * NC + lax.axis_index("c")
    row_base = wid * RPW
    sems = (sem0, sem1)

    def fire(slot, c):
        r0 = row_base + c * CB
        pltpu.sync_copy(x_hbm.at[pl.ds(r0, CB), :], idx_v.at[slot])
        for b in range(CB):
            pltpu.async_copy(
                table_hbm.at[idx_v.at[slot].at[b]],
                rows_v.at[slot].at[pl.ds(b * L, L)],
                sems[slot],
            )

    def drain(slot):
        for b in range(CB):
            pltpu.make_async_copy(
                table_hbm.at[idx_v.at[slot].at[b]],
                rows_v.at[slot].at[pl.ds(b * L, L)],
                sems[slot],
            ).wait()

    def reduce_store(slot, c):
        r0 = row_base + c * CB
        rows = rows_v.at[slot]
        for b in range(CB):
            def t_body(t, accs):
                base = b * L + UNROLL * t
                for u in range(UNROLL):
                    new = []
                    for g in range(NG):
                        ev, od = plsc.unpack(
                            rows[base + u, pl.ds(g * 32, 32)],
                            format=plsc.PackFormat.INTERLEAVED,
                        )
                        new.append(accs[2 * g] + ev)
                        new.append(accs[2 * g + 1] + od)
                    accs = tuple(new)
                return accs
            accs = lax.fori_loop(
                0, L // UNROLL, t_body,
                tuple(jnp.zeros((16,), jnp.float32) for _ in range(2 * NG)),
            )
            lane = lax.iota(jnp.int32, 16)
            brow = jnp.full((16,), b, jnp.int32)
            for g in range(NG):
                cols = lane * 2 + (32 * g)
                plsc.store_scatter(
                    out_v, [brow, cols], accs[2 * g] * jnp.float32(1.0 / L)
                )
                plsc.store_scatter(
                    out_v, [brow, cols + 1],
                    accs[2 * g + 1] * jnp.float32(1.0 / L),
                )
        pltpu.sync_copy(out_v, out_hbm.at[pl.ds(r0, CB), :])

    fire(0, 0)

    def pair_body(k, carry):
        c0 = 2 * k
        fire(1, c0 + 1)
        drain(0)
        reduce_store(0, c0)

        @pl.when(c0 + 2 < NCHUNK)
        def _():
            fire(0, c0 + 2)

        drain(1)
        reduce_store(1, c0 + 1)
        return carry

    lax.fori_loop(0, NCHUNK // 2, pair_body, 0)


@functools.partial(
    pl.kernel,
    mesh=plsc.VectorSubcoreMesh(core_axis_name="c", subcore_axis_name="s"),
    out_type=jax.ShapeDtypeStruct((B, D), jnp.float32),
    scratch_types=[
        pltpu.VMEM((2, CB, L), jnp.int32),
        pltpu.VMEM((2, CB_L, D), jnp.bfloat16),
        pltpu.VMEM((CB, D), jnp.float32),
        pltpu.SemaphoreType.DMA,
        pltpu.SemaphoreType.DMA,
    ],
    compiler_params=pltpu.CompilerParams(
        use_tc_tiling_on_sc=False, needs_layout_passes=False
    ),
)
def _pooled_lookup(x_hbm, table_hbm, out_hbm, idx_v, rows_v, out_v, sem0, sem1):
    _body(x_hbm, table_hbm, out_hbm, idx_v, rows_v, out_v, sem0, sem1)


@jax.jit
def kernel(x, table):
    return _pooled_lookup(x, table.astype(jnp.bfloat16))


# speedup vs baseline: 1.1682x; 1.1256x over previous
import functools

import jax
import jax.numpy as jnp
from jax import lax
from jax.experimental import pallas as pl
from jax.experimental.pallas import tpu as pltpu
from jax.experimental.pallas import tpu_sc as plsc

B = 16384
L = 200
D = 64
NC = 2
NS = 16
NW = NC * NS
RPW = B // NW
CB = 8
NCHUNK = RPW // CB
NG = D // 32
UNROLL = 8


def _body(x_hbm, table_hbm, out_hbm, idx_v, rows_v, out_v,
          gsem0, gsem1, isem0, isem1):
    wid = lax.axis_index("s") * NC + lax.axis_index("c")
    row_base = wid * RPW
    gsems = (gsem0, gsem1)
    isems = (isem0, isem1)

    def idx_start(slot, c):
        r0 = row_base + c * CB
        for b in range(CB):
            pltpu.async_copy(
                x_hbm.at[r0 + b],
                idx_v.at[slot].at[pl.ds(b * L, L)],
                isems[slot],
            )

    def idx_wait(slot):
        for b in range(CB):
            pltpu.make_async_copy(
                x_hbm.at[b],
                idx_v.at[slot].at[pl.ds(b * L, L)],
                isems[slot],
            ).wait()

    def fire(slot):
        pltpu.async_copy(
            table_hbm.at[idx_v.at[slot]], rows_v.at[slot], gsems[slot]
        )

    def drain(slot):
        pltpu.make_async_copy(
            table_hbm.at[idx_v.at[slot]], rows_v.at[slot], gsems[slot]
        ).wait()

    def reduce_store(slot, c):
        r0 = row_base + c * CB
        rows = rows_v.at[slot]
        for b in range(CB):
            def t_body(t, accs):
                base = b * L + UNROLL * t
                for u in range(UNROLL):
                    new = []
                    for g in range(NG):
                        ev, od = plsc.unpack(
                            rows[base + u, pl.ds(g * 32, 32)],
                            format=plsc.PackFormat.INTERLEAVED,
                        )
                        new.append(accs[2 * g] + ev)
                        new.append(accs[2 * g + 1] + od)
                    accs = tuple(new)
                return accs
            accs = lax.fori_loop(
                0, L // UNROLL, t_body,
                tuple(jnp.zeros((16,), jnp.float32) for _ in range(2 * NG)),
            )
            lane = lax.iota(jnp.int32, 16)
            brow = jnp.full((16,), b, jnp.int32)
            for g in range(NG):
                cols = lane * 2 + (32 * g)
                plsc.store_scatter(
                    out_v, [brow, cols], accs[2 * g] * jnp.float32(1.0 / L)
                )
                plsc.store_scatter(
                    out_v, [brow, cols + 1],
                    accs[2 * g + 1] * jnp.float32(1.0 / L),
                )
        pltpu.sync_copy(out_v, out_hbm.at[pl.ds(r0, CB), :])

    idx_start(0, 0)
    idx_wait(0)
    fire(0)
    idx_start(1, 1)

    def pair_body(k, carry):
        c0 = 2 * k

        idx_wait(1)
        fire(1)
        drain(0)

        @pl.when(c0 + 2 < NCHUNK)
        def _():
            idx_start(0, c0 + 2)

        reduce_store(0, c0)

        @pl.when(c0 + 2 < NCHUNK)
        def _():
            idx_wait(0)
            fire(0)

        drain(1)

        @pl.when(c0 + 3 < NCHUNK)
        def _():
            idx_start(1, c0 + 3)

        reduce_store(1, c0 + 1)
        return carry

    lax.fori_loop(0, NCHUNK // 2, pair_body, 0)


@functools.partial(
    pl.kernel,
    mesh=plsc.VectorSubcoreMesh(core_axis_name="c", subcore_axis_name="s"),
    out_type=jax.ShapeDtypeStruct((B, D), jnp.float32),
    scratch_types=[
        pltpu.VMEM((2, CB * L), jnp.int32),
        pltpu.VMEM((2, CB * L, D), jnp.bfloat16),
        pltpu.VMEM((CB, D), jnp.float32),
        pltpu.SemaphoreType.DMA,
        pltpu.SemaphoreType.DMA,
        pltpu.SemaphoreType.DMA,
        pltpu.SemaphoreType.DMA,
    ],
    compiler_params=pltpu.CompilerParams(
        use_tc_tiling_on_sc=False, needs_layout_passes=False
    ),
)
def _pooled_lookup(x_hbm, table_hbm, out_hbm, idx_v, rows_v, out_v,
                   gsem0, gsem1, isem0, isem1):
    _body(x_hbm, table_hbm, out_hbm, idx_v, rows_v, out_v,
          gsem0, gsem1, isem0, isem1)


@jax.jit
def kernel(x, table):
    return _pooled_lookup(x, table.astype(jnp.bfloat16))


# speedup vs baseline: 1.1727x; 1.0038x over previous
import functools

import jax
import jax.numpy as jnp
from jax import lax
from jax.experimental import pallas as pl
from jax.experimental.pallas import tpu as pltpu
from jax.experimental.pallas import tpu_sc as plsc

B = 16384
L = 200
D = 64
NC = 2
NS = 16
NW = NC * NS
RPW = B // NW
CB = 8
NCHUNK = RPW // CB
NG = D // 32
UNROLL = 8


def _body(x_hbm, table_hbm, out_hbm, idx_v, rows_v, out_v,
          gsem0, gsem1, isem0, isem1):
    wid = lax.axis_index("s") * NC + lax.axis_index("c")
    row_base = wid * RPW
    gsems = (gsem0, gsem1)
    isems = (isem0, isem1)

    def idx_start(slot, c):
        r0 = row_base + c * CB
        pltpu.async_copy(
            x_hbm.at[pl.ds(r0 * L, CB * L)], idx_v.at[slot], isems[slot]
        )

    def idx_wait(slot):
        pltpu.make_async_copy(
            x_hbm.at[pl.ds(0, CB * L)], idx_v.at[slot], isems[slot]
        ).wait()

    def fire(slot):
        pltpu.async_copy(
            table_hbm.at[idx_v.at[slot]], rows_v.at[slot], gsems[slot]
        )

    def drain(slot):
        pltpu.make_async_copy(
            table_hbm.at[idx_v.at[slot]], rows_v.at[slot], gsems[slot]
        ).wait()

    def reduce_store(slot, c):
        r0 = row_base + c * CB
        rows = rows_v.at[slot]
        for b in range(CB):
            def t_body(t, accs):
                base = b * L + UNROLL * t
                for u in range(UNROLL):
                    new = []
                    for g in range(NG):
                        ev, od = plsc.unpack(
                            rows[base + u, pl.ds(g * 32, 32)],
                            format=plsc.PackFormat.INTERLEAVED,
                        )
                        new.append(accs[2 * g] + ev)
                        new.append(accs[2 * g + 1] + od)
                    accs = tuple(new)
                return accs
            accs = lax.fori_loop(
                0, L // UNROLL, t_body,
                tuple(jnp.zeros((16,), jnp.float32) for _ in range(2 * NG)),
            )
            lane = lax.iota(jnp.int32, 16)
            brow = jnp.full((16,), b, jnp.int32)
            for g in range(NG):
                cols = lane * 2 + (32 * g)
                plsc.store_scatter(
                    out_v, [brow, cols], accs[2 * g] * jnp.float32(1.0 / L)
                )
                plsc.store_scatter(
                    out_v, [brow, cols + 1],
                    accs[2 * g + 1] * jnp.float32(1.0 / L),
                )
        pltpu.sync_copy(out_v, out_hbm.at[pl.ds(r0, CB), :])

    idx_start(0, 0)
    idx_wait(0)
    fire(0)
    idx_start(1, 1)

    def pair_body(k, carry):
        c0 = 2 * k

        idx_wait(1)
        fire(1)
        drain(0)

        @pl.when(c0 + 2 < NCHUNK)
        def _():
            idx_start(0, c0 + 2)

        reduce_store(0, c0)

        @pl.when(c0 + 2 < NCHUNK)
        def _():
            idx_wait(0)
            fire(0)

        drain(1)

        @pl.when(c0 + 3 < NCHUNK)
        def _():
            idx_start(1, c0 + 3)

        reduce_store(1, c0 + 1)
        return carry

    lax.fori_loop(0, NCHUNK // 2, pair_body, 0)


@functools.partial(
    pl.kernel,
    mesh=plsc.VectorSubcoreMesh(core_axis_name="c", subcore_axis_name="s"),
    out_type=jax.ShapeDtypeStruct((B, D), jnp.float32),
    scratch_types=[
        pltpu.VMEM((2, CB * L), jnp.int32),
        pltpu.VMEM((2, CB * L, D), jnp.bfloat16),
        pltpu.VMEM((CB, D), jnp.float32),
        pltpu.SemaphoreType.DMA,
        pltpu.SemaphoreType.DMA,
        pltpu.SemaphoreType.DMA,
        pltpu.SemaphoreType.DMA,
    ],
    compiler_params=pltpu.CompilerParams(
        use_tc_tiling_on_sc=False, needs_layout_passes=False
    ),
)
def _pooled_lookup(x_hbm, table_hbm, out_hbm, idx_v, rows_v, out_v,
                   gsem0, gsem1, isem0, isem1):
    _body(x_hbm, table_hbm, out_hbm, idx_v, rows_v, out_v,
          gsem0, gsem1, isem0, isem1)


@jax.jit
def kernel(x, table):
    return _pooled_lookup(x.reshape(B * L), table.astype(jnp.bfloat16))
